# baseline (speedup 1.0000x reference)
import functools

import jax
import jax.numpy as jnp
from jax import lax
from jax.experimental import pallas as pl
from jax.experimental.pallas import tpu as pltpu
from jax.experimental.pallas import tpu_sc as plsc

NC = 2
NS = 16
L = 16
NW = NC * NS

B = 16384
D = 32
BPW = B // NW
CH = 128
NCH = BPW // CH

_mesh = plsc.VectorSubcoreMesh(
    core_axis_name="c", subcore_axis_name="s", num_cores=NC, num_subcores=NS
)


@functools.partial(
    pl.kernel,
    out_type=jax.ShapeDtypeStruct((B, D), jnp.float32),
    mesh=_mesh,
    scratch_types=[
        pltpu.VMEM((NCH, CH), jnp.int32),
        pltpu.VMEM((NCH, CH), jnp.int32),
        pltpu.VMEM((BPW, D), jnp.float32),
        pltpu.VMEM((BPW, D), jnp.float32),
        pltpu.SemaphoreType.DMA,
    ],
    compiler_params=pltpu.CompilerParams(use_tc_tiling_on_sc=False),
)
def _gmf_sc(users_hbm, items_hbm, ut_hbm, it_hbm, out_hbm,
            uidx, iidx, urows, irows, sem):
    wid = lax.axis_index("s") * NC + lax.axis_index("c")
    base = wid * BPW

    pltpu.sync_copy(users_hbm.at[wid], uidx)
    pltpu.sync_copy(items_hbm.at[wid], iidx)

    copies = []
    for c in range(NCH):
        copies.append(
            pltpu.async_copy(ut_hbm.at[uidx.at[c]],
                             urows.at[pl.ds(c * CH, CH)], sem))
        copies.append(
            pltpu.async_copy(it_hbm.at[iidx.at[c]],
                             irows.at[pl.ds(c * CH, CH)], sem))
    for cp in copies:
        cp.wait()

    def body(r, _):
        for h in range(D // L):
            sl = pl.ds(h * L, L)
            urows[r, sl] = urows[r, sl] * irows[r, sl]
        return ()

    lax.fori_loop(0, BPW, body, (), unroll=4)

    pltpu.sync_copy(urows, out_hbm.at[pl.ds(base, BPW)])


def kernel(x, user_table, item_table):
    xi = x.astype(jnp.int32)
    users = xi[:, 0].reshape(NW, NCH, CH)
    items = xi[:, 1].reshape(NW, NCH, CH)
    return _gmf_sc(users, items, user_table, item_table)
